# baseline (speedup 1.0000x reference)
import functools
import jax
import jax.numpy as jnp
from jax import lax
from jax.experimental import pallas as pl
from jax.experimental.pallas import tpu as pltpu
from jax.experimental.pallas import tpu_sc as plsc

B, D, O, E = 8192, 768, 768, 8
NBUK = 64
GP = 128
CAPP = B + 4096
NBLK = CAPP // GP
NBLKP = 128
NC, NS, L = 2, 16, 16
NW = NC * NS
TPT = B // NW
TBG = 512
NBG = B // TBG
XCH = 64
NCH = TPT // XCH


def _gate_body(x_ref, wg_ref, bg_ref,
               pid_ref, wa_ref, wb_ref, prefix_ref, ps_ref,
               bea_ref, beb_ref, vcnt_ref, acc_v):
    i = pl.program_id(0)
    x = x_ref[...]
    logits = lax.dot_general(
        x, wg_ref[...], (((1,), (0,)), ((), ())),
        preferred_element_type=jnp.float32,
    ) + bg_ref[...][None, :]
    col = lax.broadcasted_iota(jnp.int32, (TBG, E), 1)
    m0 = jnp.max(logits, axis=1, keepdims=True)
    i0 = jnp.min(jnp.where(logits == m0, col, E), axis=1, keepdims=True)
    masked = jnp.where(col == i0, -jnp.inf, logits)
    m1 = jnp.max(masked, axis=1, keepdims=True)
    i1 = jnp.min(jnp.where(masked == m1, col, E), axis=1, keepdims=True)
    d = jnp.exp(m1 - m0)
    p0 = 1.0 / (1.0 + d)
    p1 = d / (1.0 + d)
    i0s, i1s = i0[:, 0], i1[:, 0]
    p0s, p1s = p0[:, 0], p1[:, 0]
    a = jnp.minimum(i0s, i1s)
    bmx = jnp.maximum(i0s, i1s)
    pid = a * E + bmx
    pid_ref[...] = pid
    first_is_a = i0s < i1s
    wa_ref[...] = jnp.where(first_is_a, p0s, p1s)
    wb_ref[...] = jnp.where(first_is_a, p1s, p0s)

    buk = lax.broadcasted_iota(jnp.int32, (TBG, NBUK), 1)
    oh = (pid[:, None] == buk).astype(jnp.int32)
    half = lax.broadcasted_iota(jnp.int32, (TBG, NBUK), 0) < (TBG // 2)
    h0 = jnp.sum(jnp.where(half, oh, 0), axis=0)
    h1 = jnp.sum(jnp.where(half, 0, oh), axis=0)

    @pl.when(i == 0)
    def _():
        acc_v[...] = jnp.zeros((NBUK,), jnp.int32)

    prev = acc_v[...]
    prefix_ref[0, 0, :] = prev
    prefix_ref[0, 1, :] = prev + h0
    acc_v[...] = prev + h0 + h1

    @pl.when(i == NBG - 1)
    def _():
        counts = prev + h0 + h1
        padded = ((counts + GP - 1) // GP) * GP
        r64 = lax.broadcasted_iota(jnp.int32, (NBUK, NBUK), 0)
        c64 = lax.broadcasted_iota(jnp.int32, (NBUK, NBUK), 1)
        tri = (r64 <= c64).astype(jnp.float32)
        ends = lax.dot_general(
            padded.astype(jnp.float32)[None, :], tri,
            (((1,), (0,)), ((), ())),
            precision=jax.lax.Precision.HIGHEST,
            preferred_element_type=jnp.float32,
        )[0].astype(jnp.int32)
        ps = ends - padded
        ps_ref[...] = ps
        blk = lax.broadcasted_iota(jnp.int32, (NBLKP,), 0) * GP
        below = (ends[None, :] <= blk[:, None]).astype(jnp.int32)
        bk = jnp.minimum(jnp.sum(below, axis=1), NBUK - 1)
        sel = bk[:, None] == lax.broadcasted_iota(jnp.int32, (NBLKP, NBUK), 1)
        c_at = jnp.sum(jnp.where(sel, counts[None, :], 0), axis=1)
        ps_at = jnp.sum(jnp.where(sel, ps[None, :], 0), axis=1)
        bea_ref[...] = bk // E
        beb_ref[...] = bk % E
        vcnt_ref[...] = jnp.clip(c_at - (blk - ps_at), 0, GP)


def _gate(x, Wg, bg):
    return pl.pallas_call(
        _gate_body,
        grid=(NBG,),
        in_specs=[
            pl.BlockSpec((TBG, D), lambda i: (i, 0)),
            pl.BlockSpec((D, E), lambda i: (0, 0)),
            pl.BlockSpec((E,), lambda i: (0,)),
        ],
        out_specs=[
            pl.BlockSpec((TBG,), lambda i: (i,)),
            pl.BlockSpec((TBG,), lambda i: (i,)),
            pl.BlockSpec((TBG,), lambda i: (i,)),
            pl.BlockSpec((1, 2, NBUK), lambda i: (i, 0, 0)),
            pl.BlockSpec((NBUK,), lambda i: (0,)),
            pl.BlockSpec((NBLKP,), lambda i: (0,)),
            pl.BlockSpec((NBLKP,), lambda i: (0,)),
            pl.BlockSpec((NBLKP,), lambda i: (0,)),
        ],
        out_shape=[
            jax.ShapeDtypeStruct((B,), jnp.int32),
            jax.ShapeDtypeStruct((B,), jnp.float32),
            jax.ShapeDtypeStruct((B,), jnp.float32),
            jax.ShapeDtypeStruct((NBG, 2, NBUK), jnp.int32),
            jax.ShapeDtypeStruct((NBUK,), jnp.int32),
            jax.ShapeDtypeStruct((NBLKP,), jnp.int32),
            jax.ShapeDtypeStruct((NBLKP,), jnp.int32),
            jax.ShapeDtypeStruct((NBLKP,), jnp.int32),
        ],
        scratch_shapes=[pltpu.VMEM((NBUK,), jnp.int32)],
    )(x, Wg, bg)


def _route_body(pid_hbm, wa_hbm, wb_hbm, start_hbm,
                tok_hbm, swa_hbm, swb_hbm, pos_hbm,
                pid_v, wa_v, wb_v, start_v, cnt_v, pos_v,
                idx_v, tok_v, tmp_v, sem_s):
    c = lax.axis_index("c")
    s = lax.axis_index("s")
    wid = s * NC + c
    base = wid * TPT
    iota = lax.iota(jnp.int32, L)

    pltpu.sync_copy(pid_hbm.at[pl.ds(base, TPT)], pid_v)
    pltpu.sync_copy(wa_hbm.at[pl.ds(base, TPT)], wa_v)
    pltpu.sync_copy(wb_hbm.at[pl.ds(base, TPT)], wb_v)
    pltpu.sync_copy(start_hbm.at[wid], start_v)

    for k in range(NBUK // L):
        cnt_v[pl.ds(k * L, L)] = jnp.zeros((L,), jnp.int32)

    for ci in range(NCH):
        for k in range(XCH // L):
            j = ci * XCH + k * L
            p = pid_v[pl.ds(j, L)]
            sk, sv = plsc.sort_key_val(p, iota)
            tmp_v[...] = sk
            prevk = plsc.load_gather(tmp_v, [jnp.maximum(iota - 1, 0)])
            nextk = plsc.load_gather(tmp_v, [jnp.minimum(iota + 1, L - 1)])
            is_start = (iota == 0) | (sk != prevk)
            is_end = (iota == L - 1) | (sk != nextk)
            lb = plsc.cummax(jnp.where(is_start, iota, 0))
            rank = iota - lb
            cbase = plsc.load_gather(cnt_v, [sk])
            sbase = plsc.load_gather(start_v, [sk])
            slot = sbase + cbase + rank
            plsc.store_scatter(cnt_v, [sk], cbase + rank + 1, mask=is_end)
            plsc.store_scatter(tmp_v, [sv], slot)
            slot_t = tmp_v[...]
            pos_v[pl.ds(j, L)] = slot_t
            idx_v[ci, pl.ds(k * L, L)] = slot_t
            tok_v[ci, pl.ds(k * L, L)] = base + j + iota

    scats = []
    for ci in range(NCH):
        scats.append(pltpu.async_copy(
            tok_v.at[ci], tok_hbm.at[idx_v.at[ci]], sem_s))
        scats.append(pltpu.async_copy(
            wa_v.at[pl.ds(ci * XCH, XCH)], swa_hbm.at[idx_v.at[ci]], sem_s))
        scats.append(pltpu.async_copy(
            wb_v.at[pl.ds(ci * XCH, XCH)], swb_hbm.at[idx_v.at[ci]], sem_s))
    pltpu.sync_copy(pos_v, pos_hbm.at[pl.ds(base, TPT)])
    for d_ in scats:
        d_.wait()


def _route(pid, wa, wb, start_all):
    mesh = plsc.VectorSubcoreMesh(core_axis_name="c", subcore_axis_name="s")
    f = pl.kernel(
        _route_body,
        out_type=[
            jax.ShapeDtypeStruct((CAPP,), jnp.int32),
            jax.ShapeDtypeStruct((CAPP,), jnp.float32),
            jax.ShapeDtypeStruct((CAPP,), jnp.float32),
            jax.ShapeDtypeStruct((B,), jnp.int32),
        ],
        mesh=mesh,
        scratch_types=[
            pltpu.VMEM((TPT,), jnp.int32),
            pltpu.VMEM((TPT,), jnp.float32),
            pltpu.VMEM((TPT,), jnp.float32),
            pltpu.VMEM((NBUK,), jnp.int32),
            pltpu.VMEM((NBUK,), jnp.int32),
            pltpu.VMEM((TPT,), jnp.int32),
            pltpu.VMEM((NCH, XCH), jnp.int32),
            pltpu.VMEM((NCH, XCH), jnp.int32),
            pltpu.VMEM((L,), jnp.int32),
            pltpu.SemaphoreType.DMA,
        ],
        compiler_params=pltpu.CompilerParams(needs_layout_passes=False),
    )
    return f(pid, wa, wb, start_all)


SPT = CAPP // NW
GCH = SPT // NCH
DH = D // 2

def _gatherx_body(xi_hbm, tok_hbm, xgi_hbm, tki_v, rows_v, sem_g, sem_st):
    c = lax.axis_index("c")
    s = lax.axis_index("s")
    wid = s * NC + c
    sbase = wid * SPT
    pltpu.sync_copy(tok_hbm.at[pl.ds(sbase, SPT)], tki_v)
    for k in range(SPT // L):
        v = tki_v[pl.ds(k * L, L)]
        tki_v[pl.ds(k * L, L)] = jnp.clip(v, 0, B - 1)
    gets = [None] * NCH
    puts = [None] * NCH
    gets[0] = pltpu.async_copy(
        xi_hbm.at[tki_v.at[pl.ds(0, GCH)]], rows_v.at[0], sem_g)
    for ci in range(NCH):
        gets[ci].wait()
        puts[ci] = pltpu.async_copy(
            rows_v.at[ci % 2], xgi_hbm.at[pl.ds(sbase + ci * GCH, GCH)],
            sem_st)
        if ci + 1 < NCH:
            if ci >= 1:
                puts[ci - 1].wait()
            gets[ci + 1] = pltpu.async_copy(
                xi_hbm.at[tki_v.at[pl.ds((ci + 1) * GCH, GCH)]],
                rows_v.at[(ci + 1) % 2], sem_g)
    puts[NCH - 2].wait()
    puts[NCH - 1].wait()


def _gatherx(xi, tok):
    mesh = plsc.VectorSubcoreMesh(core_axis_name="c", subcore_axis_name="s")
    f = pl.kernel(
        _gatherx_body,
        out_type=jax.ShapeDtypeStruct((CAPP, DH), jnp.int32),
        mesh=mesh,
        scratch_types=[
            pltpu.VMEM((SPT,), jnp.int32),
            pltpu.VMEM((2, GCH, DH), jnp.int32),
            pltpu.SemaphoreType.DMA,
            pltpu.SemaphoreType.DMA,
        ],
    )
    return f(xi, tok)


def _mm_body(bea_ref, beb_ref, vcnt_ref, xg_ref, swa_ref, swb_ref,
             we_ref, be_ref, z_ref):
    i = pl.program_id(0)
    ea = bea_ref[i]
    eb = beb_ref[i]
    xgb = xg_ref[...].astype(jnp.float32)
    ya = lax.dot_general(xgb, we_ref[ea], (((1,), (0,)), ((), ())),
                         preferred_element_type=jnp.float32)
    yb = lax.dot_general(xgb, we_ref[eb], (((1,), (0,)), ((), ())),
                         preferred_element_type=jnp.float32)
    wa = swa_ref[...][:, None]
    wb = swb_ref[...][:, None]
    z = wa * (ya + be_ref[ea][None, :]) + wb * (yb + be_ref[eb][None, :])
    row = lax.broadcasted_iota(jnp.int32, (GP, O), 0)
    z_ref[...] = jnp.where(row < vcnt_ref[i], z, 0.0)


def _mm(bea, beb, vcnt, xg, swa, swb, We, be):
    grid_spec = pltpu.PrefetchScalarGridSpec(
        num_scalar_prefetch=3,
        grid=(NBLK,),
        in_specs=[
            pl.BlockSpec((GP, D), lambda i, *_: (i, 0)),
            pl.BlockSpec((GP,), lambda i, *_: (i,)),
            pl.BlockSpec((GP,), lambda i, *_: (i,)),
            pl.BlockSpec((E, D, O), lambda i, *_: (0, 0, 0)),
            pl.BlockSpec((E, O), lambda i, *_: (0, 0)),
        ],
        out_specs=pl.BlockSpec((GP, O), lambda i, *_: (i, 0)),
    )
    return pl.pallas_call(
        _mm_body,
        grid_spec=grid_spec,
        out_shape=jax.ShapeDtypeStruct((CAPP, O), jnp.float32),
    )(bea, beb, vcnt, xg, swa, swb, We, be)


def _perm_body(z_hbm, pos_hbm, out_hbm, pidx_v, rows_v, sem_g, sem_st):
    c = lax.axis_index("c")
    s = lax.axis_index("s")
    wid = s * NC + c
    base = wid * TPT
    pltpu.sync_copy(pos_hbm.at[wid], pidx_v)
    gets = [None] * NCH
    puts = [None] * NCH
    gets[0] = pltpu.async_copy(z_hbm.at[pidx_v.at[0]], rows_v.at[0], sem_g)
    for ci in range(NCH):
        gets[ci].wait()
        puts[ci] = pltpu.async_copy(
            rows_v.at[ci % 2], out_hbm.at[pl.ds(base + ci * XCH, XCH)],
            sem_st)
        if ci + 1 < NCH:
            if ci >= 1:
                puts[ci - 1].wait()
            gets[ci + 1] = pltpu.async_copy(
                z_hbm.at[pidx_v.at[ci + 1]], rows_v.at[(ci + 1) % 2], sem_g)
    puts[NCH - 2].wait()
    puts[NCH - 1].wait()


def _perm(z, pos3):
    mesh = plsc.VectorSubcoreMesh(core_axis_name="c", subcore_axis_name="s")
    f = pl.kernel(
        _perm_body,
        out_type=jax.ShapeDtypeStruct((B, O), jnp.float32),
        mesh=mesh,
        scratch_types=[
            pltpu.VMEM((NCH, XCH), jnp.int32),
            pltpu.VMEM((2, XCH, D), jnp.float32),
            pltpu.SemaphoreType.DMA,
            pltpu.SemaphoreType.DMA,
        ],
    )
    return f(z, pos3)


def kernel(insample_y, Wg, bg, We, be):
    pid, wa, wb, prefix, ps, bea, beb, vcnt = _gate(insample_y, Wg, bg)
    start_all = ps[None, :] + prefix.reshape(NW, NBUK)
    xi = lax.bitcast_convert_type(
        insample_y.astype(jnp.bfloat16).reshape(B, DH, 2), jnp.int32)
    tok, swa, swb, pos = _route(pid, wa, wb, start_all)
    xgi = _gatherx(xi, tok)
    xgb = lax.bitcast_convert_type(xgi, jnp.bfloat16).reshape(CAPP, D)
    z = _mm(bea, beb, vcnt, xgb, swa, swb, We, be)
    return _perm(z, pos.reshape(NW, NCH, XCH))


# speedup vs baseline: 1.4013x; 1.4013x over previous
import functools
import jax
import jax.numpy as jnp
from jax import lax
from jax.experimental import pallas as pl
from jax.experimental.pallas import tpu as pltpu
from jax.experimental.pallas import tpu_sc as plsc

B, D, O, E = 8192, 768, 768, 8
NBUK = 64
GP = 128
CAPP = B + 4096
NBLK = CAPP // GP
NBLKP = 128
NC, NS, L = 2, 16, 16
NW = NC * NS
TPT = B // NW
TBG = 512
NBG = B // TBG
XCH = 64
NCH = TPT // XCH


def _gate_body(x_ref, wg_ref, bg_ref,
               pid_ref, wa_ref, wb_ref, prefix_ref, ps_ref,
               bea_ref, beb_ref, vcnt_ref, acc_v):
    i = pl.program_id(0)
    x = x_ref[...]
    logits = lax.dot_general(
        x, wg_ref[...], (((1,), (0,)), ((), ())),
        preferred_element_type=jnp.float32,
    ) + bg_ref[...][None, :]
    col = lax.broadcasted_iota(jnp.int32, (TBG, E), 1)
    m0 = jnp.max(logits, axis=1, keepdims=True)
    i0 = jnp.min(jnp.where(logits == m0, col, E), axis=1, keepdims=True)
    masked = jnp.where(col == i0, -jnp.inf, logits)
    m1 = jnp.max(masked, axis=1, keepdims=True)
    i1 = jnp.min(jnp.where(masked == m1, col, E), axis=1, keepdims=True)
    d = jnp.exp(m1 - m0)
    p0 = 1.0 / (1.0 + d)
    p1 = d / (1.0 + d)
    i0s, i1s = i0[:, 0], i1[:, 0]
    p0s, p1s = p0[:, 0], p1[:, 0]
    a = jnp.minimum(i0s, i1s)
    bmx = jnp.maximum(i0s, i1s)
    pid = a * E + bmx
    pid_ref[...] = pid
    first_is_a = i0s < i1s
    wa_ref[...] = jnp.where(first_is_a, p0s, p1s)
    wb_ref[...] = jnp.where(first_is_a, p1s, p0s)

    buk = lax.broadcasted_iota(jnp.int32, (TBG, NBUK), 1)
    oh = (pid[:, None] == buk).astype(jnp.int32)
    half = lax.broadcasted_iota(jnp.int32, (TBG, NBUK), 0) < (TBG // 2)
    h0 = jnp.sum(jnp.where(half, oh, 0), axis=0)
    h1 = jnp.sum(jnp.where(half, 0, oh), axis=0)

    @pl.when(i == 0)
    def _():
        acc_v[...] = jnp.zeros((NBUK,), jnp.int32)

    prev = acc_v[...]
    prefix_ref[0, 0, :] = prev
    prefix_ref[0, 1, :] = prev + h0
    acc_v[...] = prev + h0 + h1

    @pl.when(i == NBG - 1)
    def _():
        counts = prev + h0 + h1
        padded = ((counts + GP - 1) // GP) * GP
        r64 = lax.broadcasted_iota(jnp.int32, (NBUK, NBUK), 0)
        c64 = lax.broadcasted_iota(jnp.int32, (NBUK, NBUK), 1)
        tri = (r64 <= c64).astype(jnp.float32)
        ends = lax.dot_general(
            padded.astype(jnp.float32)[None, :], tri,
            (((1,), (0,)), ((), ())),
            precision=jax.lax.Precision.HIGHEST,
            preferred_element_type=jnp.float32,
        )[0].astype(jnp.int32)
        ps = ends - padded
        ps_ref[...] = ps
        blk = lax.broadcasted_iota(jnp.int32, (NBLKP,), 0) * GP
        below = (ends[None, :] <= blk[:, None]).astype(jnp.int32)
        bk = jnp.minimum(jnp.sum(below, axis=1), NBUK - 1)
        sel = bk[:, None] == lax.broadcasted_iota(jnp.int32, (NBLKP, NBUK), 1)
        c_at = jnp.sum(jnp.where(sel, counts[None, :], 0), axis=1)
        ps_at = jnp.sum(jnp.where(sel, ps[None, :], 0), axis=1)
        bea_ref[...] = bk // E
        beb_ref[...] = bk % E
        vcnt_ref[...] = jnp.clip(c_at - (blk - ps_at), 0, GP)


def _gate(x, Wg, bg):
    return pl.pallas_call(
        _gate_body,
        grid=(NBG,),
        in_specs=[
            pl.BlockSpec((TBG, D), lambda i: (i, 0)),
            pl.BlockSpec((D, E), lambda i: (0, 0)),
            pl.BlockSpec((E,), lambda i: (0,)),
        ],
        out_specs=[
            pl.BlockSpec((TBG,), lambda i: (i,)),
            pl.BlockSpec((TBG,), lambda i: (i,)),
            pl.BlockSpec((TBG,), lambda i: (i,)),
            pl.BlockSpec((1, 2, NBUK), lambda i: (i, 0, 0)),
            pl.BlockSpec((NBUK,), lambda i: (0,)),
            pl.BlockSpec((NBLKP,), lambda i: (0,)),
            pl.BlockSpec((NBLKP,), lambda i: (0,)),
            pl.BlockSpec((NBLKP,), lambda i: (0,)),
        ],
        out_shape=[
            jax.ShapeDtypeStruct((B,), jnp.int32),
            jax.ShapeDtypeStruct((B,), jnp.float32),
            jax.ShapeDtypeStruct((B,), jnp.float32),
            jax.ShapeDtypeStruct((NBG, 2, NBUK), jnp.int32),
            jax.ShapeDtypeStruct((NBUK,), jnp.int32),
            jax.ShapeDtypeStruct((NBLKP,), jnp.int32),
            jax.ShapeDtypeStruct((NBLKP,), jnp.int32),
            jax.ShapeDtypeStruct((NBLKP,), jnp.int32),
        ],
        scratch_shapes=[pltpu.VMEM((NBUK,), jnp.int32)],
    )(x, Wg, bg)


def _route_body(pid_hbm, wa_hbm, wb_hbm, start_hbm,
                tok_hbm, swa_hbm, swb_hbm, pos_hbm,
                pid_v, wa_v, wb_v, start_v, cnt_v, pos_v,
                idx_v, tok_v, tmp_v, sem_s):
    c = lax.axis_index("c")
    s = lax.axis_index("s")
    wid = s * NC + c
    base = wid * TPT
    iota = lax.iota(jnp.int32, L)

    pltpu.sync_copy(pid_hbm.at[pl.ds(base, TPT)], pid_v)
    pltpu.sync_copy(wa_hbm.at[pl.ds(base, TPT)], wa_v)
    pltpu.sync_copy(wb_hbm.at[pl.ds(base, TPT)], wb_v)
    pltpu.sync_copy(start_hbm.at[wid], start_v)

    for k in range(NBUK // L):
        cnt_v[pl.ds(k * L, L)] = jnp.zeros((L,), jnp.int32)

    for ci in range(NCH):
        for k in range(XCH // L):
            j = ci * XCH + k * L
            p = pid_v[pl.ds(j, L)]
            sk, sv = plsc.sort_key_val(p, iota)
            tmp_v[...] = sk
            prevk = plsc.load_gather(tmp_v, [jnp.maximum(iota - 1, 0)])
            nextk = plsc.load_gather(tmp_v, [jnp.minimum(iota + 1, L - 1)])
            is_start = (iota == 0) | (sk != prevk)
            is_end = (iota == L - 1) | (sk != nextk)
            lb = plsc.cummax(jnp.where(is_start, iota, 0))
            rank = iota - lb
            cbase = plsc.load_gather(cnt_v, [sk])
            sbase = plsc.load_gather(start_v, [sk])
            slot = sbase + cbase + rank
            plsc.store_scatter(cnt_v, [sk], cbase + rank + 1, mask=is_end)
            plsc.store_scatter(tmp_v, [sv], slot)
            slot_t = tmp_v[...]
            pos_v[pl.ds(j, L)] = slot_t
            idx_v[ci, pl.ds(k * L, L)] = slot_t
            tok_v[ci, pl.ds(k * L, L)] = base + j + iota

    scats = []
    for ci in range(NCH):
        scats.append(pltpu.async_copy(
            tok_v.at[ci], tok_hbm.at[idx_v.at[ci]], sem_s))
        scats.append(pltpu.async_copy(
            wa_v.at[pl.ds(ci * XCH, XCH)], swa_hbm.at[idx_v.at[ci]], sem_s))
        scats.append(pltpu.async_copy(
            wb_v.at[pl.ds(ci * XCH, XCH)], swb_hbm.at[idx_v.at[ci]], sem_s))
    pltpu.sync_copy(pos_v, pos_hbm.at[pl.ds(base, TPT)])
    for d_ in scats:
        d_.wait()


def _route(pid, wa, wb, start_all):
    mesh = plsc.VectorSubcoreMesh(core_axis_name="c", subcore_axis_name="s")
    f = pl.kernel(
        _route_body,
        out_type=[
            jax.ShapeDtypeStruct((CAPP,), jnp.int32),
            jax.ShapeDtypeStruct((CAPP,), jnp.float32),
            jax.ShapeDtypeStruct((CAPP,), jnp.float32),
            jax.ShapeDtypeStruct((B,), jnp.int32),
        ],
        mesh=mesh,
        scratch_types=[
            pltpu.VMEM((TPT,), jnp.int32),
            pltpu.VMEM((TPT,), jnp.float32),
            pltpu.VMEM((TPT,), jnp.float32),
            pltpu.VMEM((NBUK,), jnp.int32),
            pltpu.VMEM((NBUK,), jnp.int32),
            pltpu.VMEM((TPT,), jnp.int32),
            pltpu.VMEM((NCH, XCH), jnp.int32),
            pltpu.VMEM((NCH, XCH), jnp.int32),
            pltpu.VMEM((L,), jnp.int32),
            pltpu.SemaphoreType.DMA,
        ],
        compiler_params=pltpu.CompilerParams(needs_layout_passes=False),
    )
    return f(pid, wa, wb, start_all)


SPT = CAPP // NW
GCH = 64
NCHG = SPT // GCH

def _gatherx_body(x_hbm, tok_hbm, xg_hbm, tki_v, rows_v, sem_g, sem_st):
    c = lax.axis_index("c")
    s = lax.axis_index("s")
    wid = s * NC + c
    sbase = wid * SPT
    pltpu.sync_copy(tok_hbm.at[pl.ds(sbase, SPT)], tki_v)
    for k in range(SPT // L):
        v = tki_v[pl.ds(k * L, L)]
        tki_v[pl.ds(k * L, L)] = jnp.clip(v, 0, B - 1)
    gets = [None] * NCHG
    puts = [None] * NCHG
    gets[0] = pltpu.async_copy(
        x_hbm.at[tki_v.at[pl.ds(0, GCH)]], rows_v.at[0], sem_g)
    for ci in range(NCHG):
        gets[ci].wait()
        puts[ci] = pltpu.async_copy(
            rows_v.at[ci % 2], xg_hbm.at[pl.ds(sbase + ci * GCH, GCH)],
            sem_st)
        if ci + 1 < NCHG:
            if ci >= 1:
                puts[ci - 1].wait()
            gets[ci + 1] = pltpu.async_copy(
                x_hbm.at[tki_v.at[pl.ds((ci + 1) * GCH, GCH)]],
                rows_v.at[(ci + 1) % 2], sem_g)
    puts[NCHG - 2].wait()
    puts[NCHG - 1].wait()


def _gatherx(x, tok):
    mesh = plsc.VectorSubcoreMesh(core_axis_name="c", subcore_axis_name="s")
    f = pl.kernel(
        _gatherx_body,
        out_type=jax.ShapeDtypeStruct((CAPP, D), jnp.float32),
        mesh=mesh,
        scratch_types=[
            pltpu.VMEM((SPT,), jnp.int32),
            pltpu.VMEM((2, GCH, D), jnp.float32),
            pltpu.SemaphoreType.DMA,
            pltpu.SemaphoreType.DMA,
        ],
    )
    return f(x, tok)


def _mm_body(bea_ref, beb_ref, vcnt_ref, xg_ref, swa_ref, swb_ref,
             we_ref, be_ref, z_ref):
    i = pl.program_id(0)
    ea = bea_ref[i]
    eb = beb_ref[i]
    xgb = xg_ref[...]
    ya = lax.dot_general(xgb, we_ref[ea], (((1,), (0,)), ((), ())),
                         preferred_element_type=jnp.float32)
    yb = lax.dot_general(xgb, we_ref[eb], (((1,), (0,)), ((), ())),
                         preferred_element_type=jnp.float32)
    wa = swa_ref[...][:, None]
    wb = swb_ref[...][:, None]
    z = wa * (ya + be_ref[ea][None, :]) + wb * (yb + be_ref[eb][None, :])
    row = lax.broadcasted_iota(jnp.int32, (GP, O), 0)
    z_ref[...] = jnp.where(row < vcnt_ref[i], z, 0.0)


def _mm(bea, beb, vcnt, xg, swa, swb, We, be):
    grid_spec = pltpu.PrefetchScalarGridSpec(
        num_scalar_prefetch=3,
        grid=(NBLK,),
        in_specs=[
            pl.BlockSpec((GP, D), lambda i, *_: (i, 0)),
            pl.BlockSpec((GP,), lambda i, *_: (i,)),
            pl.BlockSpec((GP,), lambda i, *_: (i,)),
            pl.BlockSpec((E, D, O), lambda i, *_: (0, 0, 0)),
            pl.BlockSpec((E, O), lambda i, *_: (0, 0)),
        ],
        out_specs=pl.BlockSpec((GP, O), lambda i, *_: (i, 0)),
    )
    return pl.pallas_call(
        _mm_body,
        grid_spec=grid_spec,
        out_shape=jax.ShapeDtypeStruct((CAPP, O), jnp.float32),
    )(bea, beb, vcnt, xg, swa, swb, We, be)


def _perm_body(z_hbm, pos_hbm, out_hbm, pidx_v, rows_v, sem_g, sem_st):
    c = lax.axis_index("c")
    s = lax.axis_index("s")
    wid = s * NC + c
    base = wid * TPT
    pltpu.sync_copy(pos_hbm.at[wid], pidx_v)
    gets = [None] * NCH
    puts = [None] * NCH
    gets[0] = pltpu.async_copy(z_hbm.at[pidx_v.at[0]], rows_v.at[0], sem_g)
    for ci in range(NCH):
        gets[ci].wait()
        puts[ci] = pltpu.async_copy(
            rows_v.at[ci % 2], out_hbm.at[pl.ds(base + ci * XCH, XCH)],
            sem_st)
        if ci + 1 < NCH:
            if ci >= 1:
                puts[ci - 1].wait()
            gets[ci + 1] = pltpu.async_copy(
                z_hbm.at[pidx_v.at[ci + 1]], rows_v.at[(ci + 1) % 2], sem_g)
    puts[NCH - 2].wait()
    puts[NCH - 1].wait()


def _perm(z, pos3):
    mesh = plsc.VectorSubcoreMesh(core_axis_name="c", subcore_axis_name="s")
    f = pl.kernel(
        _perm_body,
        out_type=jax.ShapeDtypeStruct((B, O), jnp.float32),
        mesh=mesh,
        scratch_types=[
            pltpu.VMEM((NCH, XCH), jnp.int32),
            pltpu.VMEM((2, XCH, D), jnp.float32),
            pltpu.SemaphoreType.DMA,
            pltpu.SemaphoreType.DMA,
        ],
    )
    return f(z, pos3)


def kernel(insample_y, Wg, bg, We, be):
    pid, wa, wb, prefix, ps, bea, beb, vcnt = _gate(insample_y, Wg, bg)
    start_all = ps[None, :] + prefix.reshape(NW, NBUK)
    tok, swa, swb, pos = _route(pid, wa, wb, start_all)
    xg = _gatherx(insample_y, tok)
    z = _mm(bea, beb, vcnt, xg, swa, swb, We, be)
    return _perm(z, pos.reshape(NW, NCH, XCH))


# speedup vs baseline: 1.8541x; 1.3231x over previous
import functools
import jax
import jax.numpy as jnp
from jax import lax
from jax.experimental import pallas as pl
from jax.experimental.pallas import tpu as pltpu
from jax.experimental.pallas import tpu_sc as plsc

B, D, O, E = 8192, 768, 768, 8
NBUK = 64
GP = 128
CAPP = B + 4096
NBLK = CAPP // GP
NBLKP = 128
NC, NS, L = 2, 16, 16
NW = NC * NS
TPT = B // NW
TBG = 512
NBG = B // TBG
XCH = 64
NCH = TPT // XCH


def _gate_body(x_ref, wg_ref, bg_ref,
               pid_ref, wa_ref, wb_ref, prefix_ref, ps_ref,
               bea_ref, beb_ref, vcnt_ref, acc_v):
    i = pl.program_id(0)
    x = x_ref[...]
    logits = lax.dot_general(
        x, wg_ref[...], (((1,), (0,)), ((), ())),
        preferred_element_type=jnp.float32,
    ) + bg_ref[...][None, :]
    col = lax.broadcasted_iota(jnp.int32, (TBG, E), 1)
    m0 = jnp.max(logits, axis=1, keepdims=True)
    i0 = jnp.min(jnp.where(logits == m0, col, E), axis=1, keepdims=True)
    masked = jnp.where(col == i0, -jnp.inf, logits)
    m1 = jnp.max(masked, axis=1, keepdims=True)
    i1 = jnp.min(jnp.where(masked == m1, col, E), axis=1, keepdims=True)
    d = jnp.exp(m1 - m0)
    p0 = 1.0 / (1.0 + d)
    p1 = d / (1.0 + d)
    i0s, i1s = i0[:, 0], i1[:, 0]
    p0s, p1s = p0[:, 0], p1[:, 0]
    a = jnp.minimum(i0s, i1s)
    bmx = jnp.maximum(i0s, i1s)
    pid = a * E + bmx
    pid_ref[...] = pid
    first_is_a = i0s < i1s
    wa_ref[...] = jnp.where(first_is_a, p0s, p1s)
    wb_ref[...] = jnp.where(first_is_a, p1s, p0s)

    buk = lax.broadcasted_iota(jnp.int32, (TBG, NBUK), 1)
    oh = (pid[:, None] == buk).astype(jnp.int32)
    half = lax.broadcasted_iota(jnp.int32, (TBG, NBUK), 0) < (TBG // 2)
    h0 = jnp.sum(jnp.where(half, oh, 0), axis=0)
    h1 = jnp.sum(jnp.where(half, 0, oh), axis=0)

    @pl.when(i == 0)
    def _():
        acc_v[...] = jnp.zeros((NBUK,), jnp.int32)

    prev = acc_v[...]
    prefix_ref[0, 0, :] = prev
    prefix_ref[0, 1, :] = prev + h0
    acc_v[...] = prev + h0 + h1

    @pl.when(i == NBG - 1)
    def _():
        counts = prev + h0 + h1
        padded = ((counts + GP - 1) // GP) * GP
        r64 = lax.broadcasted_iota(jnp.int32, (NBUK, NBUK), 0)
        c64 = lax.broadcasted_iota(jnp.int32, (NBUK, NBUK), 1)
        tri = (r64 <= c64).astype(jnp.float32)
        ends = lax.dot_general(
            padded.astype(jnp.float32)[None, :], tri,
            (((1,), (0,)), ((), ())),
            precision=jax.lax.Precision.HIGHEST,
            preferred_element_type=jnp.float32,
        )[0].astype(jnp.int32)
        ps = ends - padded
        ps_ref[...] = ps
        blk = lax.broadcasted_iota(jnp.int32, (NBLKP,), 0) * GP
        below = (ends[None, :] <= blk[:, None]).astype(jnp.int32)
        bk = jnp.minimum(jnp.sum(below, axis=1), NBUK - 1)
        sel = bk[:, None] == lax.broadcasted_iota(jnp.int32, (NBLKP, NBUK), 1)
        c_at = jnp.sum(jnp.where(sel, counts[None, :], 0), axis=1)
        ps_at = jnp.sum(jnp.where(sel, ps[None, :], 0), axis=1)
        bea_ref[...] = bk // E
        beb_ref[...] = bk % E
        vcnt_ref[...] = jnp.clip(c_at - (blk - ps_at), 0, GP)


def _gate(x, Wg, bg):
    return pl.pallas_call(
        _gate_body,
        grid=(NBG,),
        in_specs=[
            pl.BlockSpec((TBG, D), lambda i: (i, 0)),
            pl.BlockSpec((D, E), lambda i: (0, 0)),
            pl.BlockSpec((E,), lambda i: (0,)),
        ],
        out_specs=[
            pl.BlockSpec((TBG,), lambda i: (i,)),
            pl.BlockSpec((TBG,), lambda i: (i,)),
            pl.BlockSpec((TBG,), lambda i: (i,)),
            pl.BlockSpec((1, 2, NBUK), lambda i: (i, 0, 0)),
            pl.BlockSpec((NBUK,), lambda i: (0,)),
            pl.BlockSpec((NBLKP,), lambda i: (0,)),
            pl.BlockSpec((NBLKP,), lambda i: (0,)),
            pl.BlockSpec((NBLKP,), lambda i: (0,)),
        ],
        out_shape=[
            jax.ShapeDtypeStruct((B,), jnp.int32),
            jax.ShapeDtypeStruct((B,), jnp.float32),
            jax.ShapeDtypeStruct((B,), jnp.float32),
            jax.ShapeDtypeStruct((NBG, 2, NBUK), jnp.int32),
            jax.ShapeDtypeStruct((NBUK,), jnp.int32),
            jax.ShapeDtypeStruct((NBLKP,), jnp.int32),
            jax.ShapeDtypeStruct((NBLKP,), jnp.int32),
            jax.ShapeDtypeStruct((NBLKP,), jnp.int32),
        ],
        scratch_shapes=[pltpu.VMEM((NBUK,), jnp.int32)],
    )(x, Wg, bg)


def _route_body(pid_hbm, wa_hbm, wb_hbm, start_hbm,
                tok_hbm, swa_hbm, swb_hbm, pos_hbm,
                pid_v, wa_v, wb_v, start_v, cnt_v, pos_v,
                idx_v, tok_v, tmp_v, sem_s):
    c = lax.axis_index("c")
    s = lax.axis_index("s")
    wid = s * NC + c
    base = wid * TPT
    iota = lax.iota(jnp.int32, L)

    pltpu.sync_copy(pid_hbm.at[pl.ds(base, TPT)], pid_v)
    pltpu.sync_copy(wa_hbm.at[pl.ds(base, TPT)], wa_v)
    pltpu.sync_copy(wb_hbm.at[pl.ds(base, TPT)], wb_v)
    pltpu.sync_copy(start_hbm.at[wid], start_v)

    for k in range(NBUK // L):
        cnt_v[pl.ds(k * L, L)] = jnp.zeros((L,), jnp.int32)

    for ci in range(NCH):
        for k in range(XCH // L):
            j = ci * XCH + k * L
            p = pid_v[pl.ds(j, L)]
            sk, sv = plsc.sort_key_val(p, iota)
            tmp_v[...] = sk
            prevk = plsc.load_gather(tmp_v, [jnp.maximum(iota - 1, 0)])
            nextk = plsc.load_gather(tmp_v, [jnp.minimum(iota + 1, L - 1)])
            is_start = (iota == 0) | (sk != prevk)
            is_end = (iota == L - 1) | (sk != nextk)
            lb = plsc.cummax(jnp.where(is_start, iota, 0))
            rank = iota - lb
            cbase = plsc.load_gather(cnt_v, [sk])
            sbase = plsc.load_gather(start_v, [sk])
            slot = sbase + cbase + rank
            plsc.store_scatter(cnt_v, [sk], cbase + rank + 1, mask=is_end)
            plsc.store_scatter(tmp_v, [sv], slot)
            slot_t = tmp_v[...]
            pos_v[pl.ds(j, L)] = slot_t
            idx_v[ci, pl.ds(k * L, L)] = slot_t
            tok_v[ci, pl.ds(k * L, L)] = base + j + iota

    scats = []
    for ci in range(NCH):
        scats.append(pltpu.async_copy(
            tok_v.at[ci], tok_hbm.at[idx_v.at[ci]], sem_s))
        scats.append(pltpu.async_copy(
            wa_v.at[pl.ds(ci * XCH, XCH)], swa_hbm.at[idx_v.at[ci]], sem_s))
        scats.append(pltpu.async_copy(
            wb_v.at[pl.ds(ci * XCH, XCH)], swb_hbm.at[idx_v.at[ci]], sem_s))
    pltpu.sync_copy(pos_v, pos_hbm.at[pl.ds(base, TPT)])
    for d_ in scats:
        d_.wait()


def _route(pid, wa, wb, start_all):
    mesh = plsc.VectorSubcoreMesh(core_axis_name="c", subcore_axis_name="s")
    f = pl.kernel(
        _route_body,
        out_type=[
            jax.ShapeDtypeStruct((CAPP,), jnp.int32),
            jax.ShapeDtypeStruct((CAPP,), jnp.float32),
            jax.ShapeDtypeStruct((CAPP,), jnp.float32),
            jax.ShapeDtypeStruct((B,), jnp.int32),
        ],
        mesh=mesh,
        scratch_types=[
            pltpu.VMEM((TPT,), jnp.int32),
            pltpu.VMEM((TPT,), jnp.float32),
            pltpu.VMEM((TPT,), jnp.float32),
            pltpu.VMEM((NBUK,), jnp.int32),
            pltpu.VMEM((NBUK,), jnp.int32),
            pltpu.VMEM((TPT,), jnp.int32),
            pltpu.VMEM((NCH, XCH), jnp.int32),
            pltpu.VMEM((NCH, XCH), jnp.int32),
            pltpu.VMEM((L,), jnp.int32),
            pltpu.SemaphoreType.DMA,
        ],
        compiler_params=pltpu.CompilerParams(needs_layout_passes=False),
    )
    return f(pid, wa, wb, start_all)


SPT = CAPP // NW
GCH = 64
NCHG = SPT // GCH

def _gatherx_body(x_hbm, tok_hbm, xg_hbm, tki_v, rows_v, sem_g, sem_st):
    c = lax.axis_index("c")
    s = lax.axis_index("s")
    wid = s * NC + c
    sbase = wid * SPT
    pltpu.sync_copy(tok_hbm.at[wid], tki_v)
    for ci in range(NCHG):
        for k in range(GCH // L):
            v = tki_v[ci, pl.ds(k * L, L)]
            tki_v[ci, pl.ds(k * L, L)] = jnp.clip(v, 0, B - 1)
    gets = [None] * NCHG
    puts = [None] * NCHG
    gets[0] = pltpu.async_copy(
        x_hbm.at[tki_v.at[0]], rows_v.at[0], sem_g)
    for ci in range(NCHG):
        gets[ci].wait()
        puts[ci] = pltpu.async_copy(
            rows_v.at[ci % 2], xg_hbm.at[pl.ds(sbase + ci * GCH, GCH)],
            sem_st)
        if ci + 1 < NCHG:
            if ci >= 1:
                puts[ci - 1].wait()
            gets[ci + 1] = pltpu.async_copy(
                x_hbm.at[tki_v.at[ci + 1]], rows_v.at[(ci + 1) % 2], sem_g)
    puts[NCHG - 2].wait()
    puts[NCHG - 1].wait()


def _gatherx(x, tok):
    mesh = plsc.VectorSubcoreMesh(core_axis_name="c", subcore_axis_name="s")
    f = pl.kernel(
        _gatherx_body,
        out_type=jax.ShapeDtypeStruct((CAPP, D), jnp.float32),
        mesh=mesh,
        scratch_types=[
            pltpu.VMEM((NCHG, GCH), jnp.int32),
            pltpu.VMEM((2, GCH, D), jnp.float32),
            pltpu.SemaphoreType.DMA,
            pltpu.SemaphoreType.DMA,
        ],
    )
    return f(x, tok.reshape(NW, NCHG, GCH))


def _mm_body(bea_ref, beb_ref, vcnt_ref, xg_ref, swa_ref, swb_ref,
             we_ref, be_ref, z_ref):
    i = pl.program_id(0)
    ea = bea_ref[i]
    eb = beb_ref[i]
    xgb = xg_ref[...]
    ya = lax.dot_general(xgb, we_ref[ea], (((1,), (0,)), ((), ())),
                         preferred_element_type=jnp.float32)
    yb = lax.dot_general(xgb, we_ref[eb], (((1,), (0,)), ((), ())),
                         preferred_element_type=jnp.float32)
    wa = swa_ref[...][:, None]
    wb = swb_ref[...][:, None]
    z = wa * (ya + be_ref[ea][None, :]) + wb * (yb + be_ref[eb][None, :])
    row = lax.broadcasted_iota(jnp.int32, (GP, O), 0)
    z_ref[...] = jnp.where(row < vcnt_ref[i], z, 0.0)


def _mm(bea, beb, vcnt, xg, swa, swb, We, be):
    grid_spec = pltpu.PrefetchScalarGridSpec(
        num_scalar_prefetch=3,
        grid=(NBLK,),
        in_specs=[
            pl.BlockSpec((GP, D), lambda i, *_: (i, 0)),
            pl.BlockSpec((GP,), lambda i, *_: (i,)),
            pl.BlockSpec((GP,), lambda i, *_: (i,)),
            pl.BlockSpec((E, D, O), lambda i, *_: (0, 0, 0)),
            pl.BlockSpec((E, O), lambda i, *_: (0, 0)),
        ],
        out_specs=pl.BlockSpec((GP, O), lambda i, *_: (i, 0)),
    )
    return pl.pallas_call(
        _mm_body,
        grid_spec=grid_spec,
        out_shape=jax.ShapeDtypeStruct((CAPP, O), jnp.float32),
    )(bea, beb, vcnt, xg, swa, swb, We, be)


def _perm_body(z_hbm, pos_hbm, out_hbm, pidx_v, rows_v, sem_g, sem_st):
    c = lax.axis_index("c")
    s = lax.axis_index("s")
    wid = s * NC + c
    base = wid * TPT
    pltpu.sync_copy(pos_hbm.at[wid], pidx_v)
    gets = [None] * NCH
    puts = [None] * NCH
    gets[0] = pltpu.async_copy(z_hbm.at[pidx_v.at[0]], rows_v.at[0], sem_g)
    for ci in range(NCH):
        gets[ci].wait()
        puts[ci] = pltpu.async_copy(
            rows_v.at[ci % 2], out_hbm.at[pl.ds(base + ci * XCH, XCH)],
            sem_st)
        if ci + 1 < NCH:
            if ci >= 1:
                puts[ci - 1].wait()
            gets[ci + 1] = pltpu.async_copy(
                z_hbm.at[pidx_v.at[ci + 1]], rows_v.at[(ci + 1) % 2], sem_g)
    puts[NCH - 2].wait()
    puts[NCH - 1].wait()


def _perm(z, pos3):
    mesh = plsc.VectorSubcoreMesh(core_axis_name="c", subcore_axis_name="s")
    f = pl.kernel(
        _perm_body,
        out_type=jax.ShapeDtypeStruct((B, O), jnp.float32),
        mesh=mesh,
        scratch_types=[
            pltpu.VMEM((NCH, XCH), jnp.int32),
            pltpu.VMEM((2, XCH, D), jnp.float32),
            pltpu.SemaphoreType.DMA,
            pltpu.SemaphoreType.DMA,
        ],
    )
    return f(z, pos3)


def kernel(insample_y, Wg, bg, We, be):
    pid, wa, wb, prefix, ps, bea, beb, vcnt = _gate(insample_y, Wg, bg)
    start_all = ps[None, :] + prefix.reshape(NW, NBUK)
    tok, swa, swb, pos = _route(pid, wa, wb, start_all)
    xg = _gatherx(insample_y, tok)
    z = _mm(bea, beb, vcnt, xg, swa, swb, We, be)
    return _perm(z, pos.reshape(NW, NCH, XCH))


# speedup vs baseline: 6.2886x; 3.3918x over previous
import functools
import jax
import jax.numpy as jnp
from jax.experimental import pallas as pl
from jax.experimental.pallas import tpu as pltpu

B, D, O, E, K = 8192, 768, 768, 8, 2
TB = 1024


def _fused_body(x_ref, wg_ref, bg_ref, we_ref, be_ref, out_ref):
    x = x_ref[...]
    logits = jax.lax.dot_general(
        x, wg_ref[...], (((1,), (0,)), ((), ())),
        preferred_element_type=jnp.float32,
    ) + bg_ref[...][None, :]

    col = jax.lax.broadcasted_iota(jnp.int32, (TB, E), 1)
    m0 = jnp.max(logits, axis=1, keepdims=True)
    i0 = jnp.min(jnp.where(logits == m0, col, E), axis=1, keepdims=True)
    masked = jnp.where(col == i0, -jnp.inf, logits)
    m1 = jnp.max(masked, axis=1, keepdims=True)
    i1 = jnp.min(jnp.where(masked == m1, col, E), axis=1, keepdims=True)
    d = jnp.exp(m1 - m0)
    p0 = 1.0 / (1.0 + d)
    p1 = d / (1.0 + d)
    w = jnp.where(col == i0, p0, jnp.where(col == i1, p1, 0.0))

    acc = jax.lax.dot_general(
        w, be_ref[...], (((1,), (0,)), ((), ())),
        preferred_element_type=jnp.float32,
    )
    for e in range(E):
        y = jax.lax.dot_general(
            x, we_ref[e], (((1,), (0,)), ((), ())),
            preferred_element_type=jnp.float32,
        )
        acc = acc + y * w[:, e][:, None]
    out_ref[...] = acc


def kernel(insample_y, Wg, bg, We, be):
    grid = (B // TB,)
    return pl.pallas_call(
        _fused_body,
        grid=grid,
        in_specs=[
            pl.BlockSpec((TB, D), lambda i: (i, 0)),
            pl.BlockSpec((D, E), lambda i: (0, 0)),
            pl.BlockSpec((E,), lambda i: (0,)),
            pl.BlockSpec((E, D, O), lambda i: (0, 0, 0)),
            pl.BlockSpec((E, O), lambda i: (0, 0)),
        ],
        out_specs=pl.BlockSpec((TB, O), lambda i: (i, 0)),
        out_shape=jax.ShapeDtypeStruct((B, O), jnp.float32),
    )(insample_y, Wg, bg, We, be)
